# baseline (speedup 1.0000x reference)
import functools

import jax
import jax.numpy as jnp
from jax import lax
from jax.experimental import pallas as pl
from jax.experimental.pallas import tpu as pltpu
from jax.experimental.pallas import tpu_sc as plsc

NUM_SEG = 16
TOTAL_TOK = 32768
D = 1024
L = 16
NC = 2
NS = 16
NW = NC * NS
RPW = TOTAL_TOK // NW
C = 16
NCH = RPW // C
NDC = D // L
ACC = NUM_SEG * D

_mesh = plsc.VectorSubcoreMesh(core_axis_name="c", subcore_axis_name="s")


@functools.partial(
    pl.kernel,
    mesh=_mesh,
    out_type=[
        jax.ShapeDtypeStruct((NW, ACC), jnp.float32),
        jax.ShapeDtypeStruct((NW, ACC), jnp.float32),
    ],
    scratch_types=[
        pltpu.VMEM((2, C, D), jnp.float32),
        pltpu.VMEM((2, C, D), jnp.float32),
        pltpu.VMEM((RPW,), jnp.int32),
        pltpu.VMEM((ACC,), jnp.float32),
        pltpu.VMEM((ACC,), jnp.float32),
        pltpu.SemaphoreType.DMA,
        pltpu.SemaphoreType.DMA,
        pltpu.SemaphoreType.DMA,
        pltpu.SemaphoreType.DMA,
    ],
    compiler_params=pltpu.CompilerParams(needs_layout_passes=False),
)
def _seg_sumsq(inp, tgt, idx, outd, outt, bufa, buft, idxv, accd, acct,
               sa0, sa1, st0, st1):
    wid = lax.axis_index("s") * NC + lax.axis_index("c")
    base = wid * RPW
    sems_a = (sa0, sa1)
    sems_t = (st0, st1)

    pltpu.sync_copy(idx.at[pl.ds(base, RPW)], idxv)

    zero = jnp.zeros((L,), jnp.float32)
    lane = lax.iota(jnp.int32, L)

    def _z(i, carry):
        accd[pl.ds(i * L, L)] = zero
        acct[pl.ds(i * L, L)] = zero
        return carry

    lax.fori_loop(0, ACC // L, _z, 0, unroll=8)

    def _start(ch, b):
        r0 = base + ch * C
        pltpu.make_async_copy(inp.at[pl.ds(r0, C), :], bufa.at[b],
                              sems_a[b]).start()
        pltpu.make_async_copy(tgt.at[pl.ds(r0, C), :], buft.at[b],
                              sems_t[b]).start()

    def _wait(b):
        pltpu.make_async_copy(inp.at[pl.ds(base, C), :], bufa.at[b],
                              sems_a[b]).wait()
        pltpu.make_async_copy(tgt.at[pl.ds(base, C), :], buft.at[b],
                              sems_t[b]).wait()

    _start(0, 0)
    _start(1, 1)

    def _chunk(ch, b):
        _wait(b)
        vi = idxv[pl.ds(ch * L, L)]

        for r in range(C):
            vr = lax.gather(
                vi, jnp.full((L, 1), r, jnp.int32),
                dimension_numbers=lax.GatherDimensionNumbers(
                    offset_dims=(), collapsed_slice_dims=(0,),
                    start_index_map=(0,)),
                slice_sizes=(1,),
                mode=lax.GatherScatterMode.PROMISE_IN_BOUNDS)
            base_r = vr * D + lane

            def _dc(dc, carry):
                av = bufa[b, r, pl.ds(dc * L, L)]
                tv = buft[b, r, pl.ds(dc * L, L)]
                dv = av - tv
                ia = base_r + dc * L
                plsc.addupdate_scatter(accd, [ia], dv * dv)
                plsc.addupdate_scatter(acct, [ia], tv * tv)
                return carry

            lax.fori_loop(0, NDC, _dc, 0, unroll=4)

        @pl.when(ch + 2 < NCH)
        def _next():
            _start(ch + 2, b)

    def _outer(g, carry):
        _chunk(g * 2, 0)
        _chunk(g * 2 + 1, 1)
        return carry

    lax.fori_loop(0, NCH // 2, _outer, 0)

    pltpu.sync_copy(accd, outd.at[wid])
    pltpu.sync_copy(acct, outt.at[wid])


def _epi_body(pd_ref, pt_ref, o_ref):
    sd = jnp.sum(pd_ref[...], axis=0)
    st = jnp.sum(pt_ref[...], axis=0)
    dn = jnp.sqrt(sd)
    tn = jnp.maximum(jnp.sqrt(st), 1.0)
    o_ref[0, 0] = jnp.mean(dn / tn)


def _epilogue(pd, pt):
    return pl.pallas_call(
        _epi_body,
        out_specs=pl.BlockSpec(memory_space=pltpu.SMEM),
        out_shape=jax.ShapeDtypeStruct((1, 1), jnp.float32),
    )(pd, pt)


def kernel(input, target, batch_idx):
    pd, pt = _seg_sumsq(input, target, batch_idx.astype(jnp.int32))
    return _epilogue(pd, pt)[0, 0]


# speedup vs baseline: 2.8601x; 2.8601x over previous
import functools

import jax
import jax.numpy as jnp
from jax import lax
from jax.experimental import pallas as pl
from jax.experimental.pallas import tpu as pltpu
from jax.experimental.pallas import tpu_sc as plsc

NUM_SEG = 16
TOTAL_TOK = 32768
D = 1024
L = 16
NC = 2
NS = 16
NW = NC * NS
RPW = TOTAL_TOK // NW
C = 16
NCH = RPW // C
NDC = D // L
ACC = NUM_SEG * D

_mesh = plsc.VectorSubcoreMesh(core_axis_name="c", subcore_axis_name="s")


@functools.partial(
    pl.kernel,
    mesh=_mesh,
    out_type=[
        jax.ShapeDtypeStruct((NW, ACC), jnp.float32),
        jax.ShapeDtypeStruct((NW, ACC), jnp.float32),
    ],
    scratch_types=[
        pltpu.VMEM((2, C, D), jnp.float32),
        pltpu.VMEM((2, C, D), jnp.float32),
        pltpu.VMEM((RPW,), jnp.int32),
        pltpu.VMEM((ACC,), jnp.float32),
        pltpu.VMEM((ACC,), jnp.float32),
        pltpu.SemaphoreType.DMA,
        pltpu.SemaphoreType.DMA,
        pltpu.SemaphoreType.DMA,
        pltpu.SemaphoreType.DMA,
    ],
    compiler_params=pltpu.CompilerParams(needs_layout_passes=False),
)
def _seg_sumsq(inp, tgt, idx, outd, outt, bufa, buft, idxv, accd, acct,
               sa0, sa1, st0, st1):
    wid = lax.axis_index("s") * NC + lax.axis_index("c")
    base = wid * RPW
    sems_a = (sa0, sa1)
    sems_t = (st0, st1)

    pltpu.sync_copy(idx.at[pl.ds(base, RPW)], idxv)

    zero = jnp.zeros((L,), jnp.float32)
    lane = lax.iota(jnp.int32, L)

    def _z(i, carry):
        accd[pl.ds(i * L, L)] = zero
        acct[pl.ds(i * L, L)] = zero
        return carry

    lax.fori_loop(0, ACC // L, _z, 0, unroll=8)

    def _start(ch, b):
        r0 = base + ch * C
        pltpu.make_async_copy(inp.at[pl.ds(r0, C), :], bufa.at[b],
                              sems_a[b]).start()
        pltpu.make_async_copy(tgt.at[pl.ds(r0, C), :], buft.at[b],
                              sems_t[b]).start()

    def _wait(b):
        pltpu.make_async_copy(inp.at[pl.ds(base, C), :], bufa.at[b],
                              sems_a[b]).wait()
        pltpu.make_async_copy(tgt.at[pl.ds(base, C), :], buft.at[b],
                              sems_t[b]).wait()

    _start(0, 0)
    _start(1, 1)

    def _chunk(ch, b):
        _wait(b)
        vi = idxv[pl.ds(ch * C, L)]
        s0 = vi[0]
        s15 = vi[L - 1]

        @pl.when(s0 == s15)
        def _fast():
            off = s0 * D

            def _dc(dc, carry):
                ad = zero
                at = zero
                for r in range(C):
                    av = bufa[b, r, pl.ds(dc * L, L)]
                    tv = buft[b, r, pl.ds(dc * L, L)]
                    dv = av - tv
                    ad = ad + dv * dv
                    at = at + tv * tv
                accd[pl.ds(off + dc * L, L)] += ad
                acct[pl.ds(off + dc * L, L)] += at
                return carry

            lax.fori_loop(0, NDC, _dc, 0)

        @pl.when(s0 != s15)
        def _slow():
            for r in range(C):
                sr = vi[r]
                off = sr * D

                def _dc2(dc, carry):
                    av = bufa[b, r, pl.ds(dc * L, L)]
                    tv = buft[b, r, pl.ds(dc * L, L)]
                    dv = av - tv
                    accd[pl.ds(off + dc * L, L)] += dv * dv
                    acct[pl.ds(off + dc * L, L)] += tv * tv
                    return carry

                lax.fori_loop(0, NDC, _dc2, 0, unroll=4)

        @pl.when(ch + 2 < NCH)
        def _next():
            _start(ch + 2, b)

    def _outer(g, carry):
        _chunk(g * 2, 0)
        _chunk(g * 2 + 1, 1)
        return carry

    lax.fori_loop(0, NCH // 2, _outer, 0)

    pltpu.sync_copy(accd, outd.at[wid])
    pltpu.sync_copy(acct, outt.at[wid])


def _epi_body(pd_ref, pt_ref, o_ref):
    sd = jnp.sum(pd_ref[...], axis=0)
    st = jnp.sum(pt_ref[...], axis=0)
    dn = jnp.sqrt(sd)
    tn = jnp.maximum(jnp.sqrt(st), 1.0)
    o_ref[0, 0] = jnp.mean(dn / tn)


def _epilogue(pd, pt):
    return pl.pallas_call(
        _epi_body,
        out_specs=pl.BlockSpec(memory_space=pltpu.SMEM),
        out_shape=jax.ShapeDtypeStruct((1, 1), jnp.float32),
    )(pd, pt)


def kernel(input, target, batch_idx):
    pd, pt = _seg_sumsq(input, target, batch_idx.astype(jnp.int32))
    return _epilogue(pd, pt)[0, 0]
